# Initial kernel scaffold; baseline (speedup 1.0000x reference)
import jax
import jax.numpy as jnp
from jax.experimental import pallas as pl


def kernel(node_features, edge_index, W1, Wa):
    raise NotImplementedError("write your pallas kernel here")


# speedup vs baseline: 3.0755x; 3.0755x over previous
import jax
import jax.numpy as jnp
from jax import lax
from jax.experimental import pallas as pl
from jax.experimental.pallas import tpu as pltpu
from jax.experimental.pallas import tpu_sc as plsc

N = 10000
E = 160000
D = 256
NW = 32
NPW = 320
NPAD = NW * NPW
ZROWS = NPW
CHUNK = 2000
NGRP = CHUNK // 16
NCHUNK = E // CHUNK
SELCAP = CHUNK + 16
DNCAP = NPW + 23


def _node_body(x_ref, w_ref, ap_ref, wh_ref, s_ref):
    wh = jnp.dot(x_ref[...], w_ref[...], preferred_element_type=jnp.float32)
    wh_ref[...] = wh
    s_ref[...] = jnp.dot(wh, ap_ref[...], preferred_element_type=jnp.float32)


def _node_phase(x, w1t, ap):
    return pl.pallas_call(
        _node_body,
        grid=(10,),
        in_specs=[
            pl.BlockSpec((1000, D), lambda i: (i, 0)),
            pl.BlockSpec((D, D), lambda i: (0, 0)),
            pl.BlockSpec((D, 128), lambda i: (0, 0)),
        ],
        out_specs=[
            pl.BlockSpec((1000, D), lambda i: (i, 0)),
            pl.BlockSpec((1000, 128), lambda i: (i, 0)),
        ],
        out_shape=[
            jax.ShapeDtypeStruct((N, D), jnp.float32),
            jax.ShapeDtypeStruct((N, 128), jnp.float32),
        ],
    )(x, w1t, ap)


def _gat_body(src_hbm, dst_hbm, s1_hbm, s2_hbm, wh_hbm, out_hbm,
              s1_v, s2_v, src_c, dst_c, srcsel, dstsel, z_v, denom_v,
              rows_v, e_buf, dl_buf, sem):
    wid = lax.axis_index("s") * 2 + lax.axis_index("c")
    lo = wid * NPW

    pltpu.sync_copy(s1_hbm, s1_v)
    pltpu.sync_copy(s2_hbm, s2_v)

    zero16f = jnp.zeros((16,), jnp.float32)
    zero16i = jnp.zeros((16,), jnp.int32)
    unit16f = jnp.where(lax.iota(jnp.int32, 16) == 0, 1.0, 0.0)

    def _zero_sel(i, _):
        srcsel[pl.ds(i * 16, 16)] = zero16i
        dstsel[pl.ds(i * 16, 16)] = zero16i
        return 0
    lax.fori_loop(0, SELCAP // 16, _zero_sel, 0)

    def _zero_z(r, _):
        for k in range(D // 16):
            z_v[r, pl.ds(k * 16, 16)] = zero16f
        return 0
    lax.fori_loop(0, ZROWS, _zero_z, 0)

    def _zero_dn(i, _):
        denom_v[pl.ds(i * 16, 16)] = zero16f
        return 0
    lax.fori_loop(0, DNCAP // 16, _zero_dn, 0)

    def _chunk(c, _):
        base = c * CHUNK
        pltpu.sync_copy(src_hbm.at[pl.ds(base, CHUNK)], src_c)
        pltpu.sync_copy(dst_hbm.at[pl.ds(base, CHUNK)], dst_c)

        def _scan(g, nsel):
            off = g * 16
            d = dst_c[pl.ds(off, 16)]
            s = src_c[pl.ds(off, 16)]
            m = (d >= lo) & (d < lo + NPW)
            c = plsc.cumsum(m.astype(jnp.int32))
            pos = nsel + c - 1
            plsc.store_scatter(srcsel, [pos], s, mask=m)
            plsc.store_scatter(dstsel, [pos], d, mask=m)
            return nsel + c[15]

        nsel = lax.fori_loop(0, NGRP, _scan, 0)
        ngrp = (nsel + 15) // 16

        def _proc(g, _):
            goff = g * 16
            sv = srcsel[pl.ds(goff, 16)]
            dv = dstsel[pl.ds(goff, 16)]
            s1g = plsc.load_gather(s1_v, [sv])
            s2g = plsc.load_gather(s2_v, [dv])
            e16 = jnp.exp(jnp.maximum(s1g + s2g, 0.0))
            e_buf[pl.ds(0, 16)] = e16
            dl_buf[pl.ds(0, 16)] = dv - lo
            pltpu.async_copy(wh_hbm.at[srcsel.at[pl.ds(goff, 16)]],
                             rows_v, sem).wait()
            rem = nsel - goff

            def _edge(j, _):
                @pl.when(j < rem)
                def _():
                    dl = dl_buf[pl.ds(j, 16)][0]
                    ej = e_buf[pl.ds(j, 16)][0]
                    denom_v[pl.ds(dl, 16)] = (
                        denom_v[pl.ds(dl, 16)] + ej * unit16f)
                    for k in range(D // 16):
                        ko = k * 16
                        z_v[dl, pl.ds(ko, 16)] = (
                            z_v[dl, pl.ds(ko, 16)]
                            + ej * rows_v[j, pl.ds(ko, 16)])
                return 0
            lax.fori_loop(0, 16, _edge, 0)
            return 0

        lax.fori_loop(0, ngrp, _proc, 0)
        return 0

    lax.fori_loop(0, NCHUNK, _chunk, 0)

    def _norm(r, _):
        dnv = denom_v[pl.ds(r, 16)]
        inv = jnp.where(dnv > 0.0, 1.0 / dnv, 1.0)[0]
        for k in range(D // 16):
            ko = k * 16
            z_v[r, pl.ds(ko, 16)] = z_v[r, pl.ds(ko, 16)] * inv
        return 0
    lax.fori_loop(0, NPW, _norm, 0)

    pltpu.sync_copy(z_v.at[pl.ds(0, NPW)], out_hbm.at[pl.ds(lo, NPW)])


def _edge_phase(src, dst, s1, s2, wh):
    mesh = plsc.VectorSubcoreMesh(core_axis_name="c", subcore_axis_name="s")
    kern = pl.kernel(
        _gat_body,
        out_type=jax.ShapeDtypeStruct((NPAD, D), jnp.float32),
        mesh=mesh,
        scratch_types=[
            pltpu.VMEM((N,), jnp.float32),
            pltpu.VMEM((N,), jnp.float32),
            pltpu.VMEM((CHUNK,), jnp.int32),
            pltpu.VMEM((CHUNK,), jnp.int32),
            pltpu.VMEM((SELCAP,), jnp.int32),
            pltpu.VMEM((SELCAP,), jnp.int32),
            pltpu.VMEM((ZROWS, D), jnp.float32),
            pltpu.VMEM((DNCAP,), jnp.float32),
            pltpu.VMEM((16, D), jnp.float32),
            pltpu.VMEM((32,), jnp.float32),
            pltpu.VMEM((32,), jnp.int32),
            pltpu.SemaphoreType.DMA,
        ],
        compiler_params=pltpu.CompilerParams(needs_layout_passes=False),
    )
    return kern(src, dst, s1, s2, wh)


def kernel(node_features, edge_index, W1, Wa):
    src = edge_index[0]
    dst = edge_index[1]
    w1t = W1.T
    a1 = Wa[0, :D]
    a2 = Wa[0, D:]
    ap = jnp.zeros((D, 128), jnp.float32).at[:, 0].set(a1).at[:, 1].set(a2)
    wh, svec = _node_phase(node_features, w1t, ap)
    s1 = svec[:, 0]
    s2 = svec[:, 1]
    zpad = _edge_phase(src, dst, s1, s2, wh)
    return zpad[:N]


# speedup vs baseline: 3.4712x; 1.1287x over previous
import jax
import jax.numpy as jnp
from jax import lax
from jax.experimental import pallas as pl
from jax.experimental.pallas import tpu as pltpu
from jax.experimental.pallas import tpu_sc as plsc

N = 10000
E = 160000
D = 256
NW = 32
NPW = 320
NPAD = NW * NPW
ZROWS = NPW
CHUNK = 2000
NGRP = CHUNK // 16
NCHUNK = E // CHUNK
SELCAP = CHUNK + 16
DNCAP = NPW + 23


def _node_body(x_ref, w_ref, ap_ref, wh_ref, s_ref):
    wh = jnp.dot(x_ref[...], w_ref[...], preferred_element_type=jnp.float32)
    wh_ref[...] = wh
    s_ref[...] = jnp.dot(wh, ap_ref[...], preferred_element_type=jnp.float32)


def _node_phase(x, w1t, ap):
    return pl.pallas_call(
        _node_body,
        grid=(10,),
        in_specs=[
            pl.BlockSpec((1000, D), lambda i: (i, 0)),
            pl.BlockSpec((D, D), lambda i: (0, 0)),
            pl.BlockSpec((D, 128), lambda i: (0, 0)),
        ],
        out_specs=[
            pl.BlockSpec((1000, D), lambda i: (i, 0)),
            pl.BlockSpec((1000, 128), lambda i: (i, 0)),
        ],
        out_shape=[
            jax.ShapeDtypeStruct((N, D), jnp.float32),
            jax.ShapeDtypeStruct((N, 128), jnp.float32),
        ],
    )(x, w1t, ap)


def _gat_body(src_hbm, dst_hbm, s1_hbm, s2_hbm, wh_hbm, out_hbm,
              s1_v, s2_v, src_c, dst_c, srcsel, dstsel, z_v, denom_v,
              rows_v, sem):
    wid = lax.axis_index("s") * 2 + lax.axis_index("c")
    lo = wid * NPW

    pltpu.sync_copy(s1_hbm, s1_v)
    pltpu.sync_copy(s2_hbm, s2_v)

    zero16f = jnp.zeros((16,), jnp.float32)
    zero16i = jnp.zeros((16,), jnp.int32)
    unit16f = jnp.where(lax.iota(jnp.int32, 16) == 0, 1.0, 0.0)

    def _zero_sel(i, _):
        srcsel[pl.ds(i * 16, 16)] = zero16i
        dstsel[pl.ds(i * 16, 16)] = zero16i
        return 0
    lax.fori_loop(0, SELCAP // 16, _zero_sel, 0)

    def _zero_z(r, _):
        for k in range(D // 16):
            z_v[r, pl.ds(k * 16, 16)] = zero16f
        return 0
    lax.fori_loop(0, ZROWS, _zero_z, 0)

    def _zero_dn(i, _):
        denom_v[pl.ds(i * 16, 16)] = zero16f
        return 0
    lax.fori_loop(0, DNCAP // 16, _zero_dn, 0)

    def _chunk(c, _):
        base = c * CHUNK
        pltpu.sync_copy(src_hbm.at[pl.ds(base, CHUNK)], src_c)
        pltpu.sync_copy(dst_hbm.at[pl.ds(base, CHUNK)], dst_c)

        def _scan(g, nsel):
            off = g * 16
            d = dst_c[pl.ds(off, 16)]
            s = src_c[pl.ds(off, 16)]
            m = (d >= lo) & (d < lo + NPW)
            c = plsc.cumsum(m.astype(jnp.int32))
            pos = nsel + c - 1
            plsc.store_scatter(srcsel, [pos], s, mask=m)
            plsc.store_scatter(dstsel, [pos], d, mask=m)
            return nsel + c[15]

        nsel = lax.fori_loop(0, NGRP, _scan, 0)
        ngrp = (nsel + 15) // 16

        def _issue(g):
            b16 = (g & 1) * 16
            pltpu.async_copy(wh_hbm.at[srcsel.at[pl.ds(g * 16, 16)]],
                             rows_v.at[pl.ds(b16, 16)], sem.at[g & 1])

        @pl.when(ngrp > 0)
        def _():
            _issue(0)

        def _proc(g, _):
            b16 = (g & 1) * 16
            goff = g * 16
            sv = srcsel[pl.ds(goff, 16)]
            dv = dstsel[pl.ds(goff, 16)]
            s1g = plsc.load_gather(s1_v, [sv])
            s2g = plsc.load_gather(s2_v, [dv])
            valid = lax.iota(jnp.int32, 16) < (nsel - goff)
            e16 = jnp.where(valid, jnp.exp(jnp.maximum(s1g + s2g, 0.0)), 0.0)
            dl16 = jnp.where(valid, dv - lo, 0)
            pltpu.make_async_copy(wh_hbm.at[pl.ds(0, 16)],
                                  rows_v.at[pl.ds(b16, 16)],
                                  sem.at[g & 1]).wait()

            @pl.when(g + 1 < ngrp)
            def _():
                _issue(g + 1)

            for j in range(16):
                ej = e16[j]
                dl = dl16[j]
                denom_v[pl.ds(dl, 16)] = (
                    denom_v[pl.ds(dl, 16)] + ej * unit16f)
                for k in range(D // 16):
                    ko = k * 16
                    z_v[dl, pl.ds(ko, 16)] = (
                        z_v[dl, pl.ds(ko, 16)]
                        + ej * rows_v[b16 + j, pl.ds(ko, 16)])
            return 0

        lax.fori_loop(0, ngrp, _proc, 0)
        return 0

    lax.fori_loop(0, NCHUNK, _chunk, 0)

    def _norm(r, _):
        dnv = denom_v[pl.ds(r, 16)]
        inv = jnp.where(dnv > 0.0, 1.0 / dnv, 1.0)[0]
        for k in range(D // 16):
            ko = k * 16
            z_v[r, pl.ds(ko, 16)] = z_v[r, pl.ds(ko, 16)] * inv
        return 0
    lax.fori_loop(0, NPW, _norm, 0)

    pltpu.sync_copy(z_v.at[pl.ds(0, NPW)], out_hbm.at[pl.ds(lo, NPW)])


def _edge_phase(src, dst, s1, s2, wh):
    mesh = plsc.VectorSubcoreMesh(core_axis_name="c", subcore_axis_name="s")
    kern = pl.kernel(
        _gat_body,
        out_type=jax.ShapeDtypeStruct((NPAD, D), jnp.float32),
        mesh=mesh,
        scratch_types=[
            pltpu.VMEM((N,), jnp.float32),
            pltpu.VMEM((N,), jnp.float32),
            pltpu.VMEM((CHUNK,), jnp.int32),
            pltpu.VMEM((CHUNK,), jnp.int32),
            pltpu.VMEM((SELCAP,), jnp.int32),
            pltpu.VMEM((SELCAP,), jnp.int32),
            pltpu.VMEM((ZROWS, D), jnp.float32),
            pltpu.VMEM((DNCAP,), jnp.float32),
            pltpu.VMEM((32, D), jnp.float32),
            pltpu.SemaphoreType.DMA((2,)),
        ],
        compiler_params=pltpu.CompilerParams(needs_layout_passes=False),
    )
    return kern(src, dst, s1, s2, wh)


def kernel(node_features, edge_index, W1, Wa):
    src = edge_index[0]
    dst = edge_index[1]
    w1t = W1.T
    a1 = Wa[0, :D]
    a2 = Wa[0, D:]
    ap = jnp.zeros((D, 128), jnp.float32).at[:, 0].set(a1).at[:, 1].set(a2)
    wh, svec = _node_phase(node_features, w1t, ap)
    s1 = svec[:, 0]
    s2 = svec[:, 1]
    zpad = _edge_phase(src, dst, s1, s2, wh)
    return zpad[:N]


# speedup vs baseline: 4.4773x; 1.2898x over previous
import jax
import jax.numpy as jnp
from jax import lax
from jax.experimental import pallas as pl
from jax.experimental.pallas import tpu as pltpu
from jax.experimental.pallas import tpu_sc as plsc

N = 10000
E = 160000
D = 256
NW = 32
NPW = 320
NPAD = NW * NPW
ZROWS = NPW
CHUNK = 2000
NGRP = CHUNK // 16
NCHUNK = E // CHUNK
SELCAP = CHUNK + 16
DNCAP = NPW + 23


def _node_body(x_ref, w_ref, ap_ref, wh_ref, s_ref):
    wh = jnp.dot(x_ref[...], w_ref[...], preferred_element_type=jnp.float32)
    wh_ref[...] = wh
    s_ref[...] = jnp.dot(wh, ap_ref[...], preferred_element_type=jnp.float32)


def _node_phase(x, w1t, ap):
    return pl.pallas_call(
        _node_body,
        grid=(10,),
        in_specs=[
            pl.BlockSpec((1000, D), lambda i: (i, 0)),
            pl.BlockSpec((D, D), lambda i: (0, 0)),
            pl.BlockSpec((D, 128), lambda i: (0, 0)),
        ],
        out_specs=[
            pl.BlockSpec((1000, D), lambda i: (i, 0)),
            pl.BlockSpec((1000, 128), lambda i: (i, 0)),
        ],
        out_shape=[
            jax.ShapeDtypeStruct((N, D), jnp.float32),
            jax.ShapeDtypeStruct((N, 128), jnp.float32),
        ],
    )(x, w1t, ap)


def _gat_body(src_hbm, dst_hbm, s1_hbm, s2_hbm, wh_hbm, out_hbm,
              s1_v, s2_v, src_c, dst_c, srcsel, dstsel, z_v, denom_v,
              rows_v, sem):
    wid = lax.axis_index("s") * 2 + lax.axis_index("c")
    lo = wid * NPW

    pltpu.sync_copy(s1_hbm, s1_v)
    pltpu.sync_copy(s2_hbm, s2_v)

    zero16f = jnp.zeros((16,), jnp.float32)
    zero16i = jnp.zeros((16,), jnp.int32)
    unit16f = jnp.where(lax.iota(jnp.int32, 16) == 0, 1.0, 0.0)

    def _zero_sel(i, _):
        srcsel[pl.ds(i * 16, 16)] = zero16i
        dstsel[pl.ds(i * 16, 16)] = zero16i
        return 0
    lax.fori_loop(0, SELCAP // 16, _zero_sel, 0)

    def _zero_z(r, _):
        for k in range(D // 16):
            z_v[r, pl.ds(k * 16, 16)] = zero16f
        return 0
    lax.fori_loop(0, ZROWS, _zero_z, 0)

    def _zero_dn(i, _):
        denom_v[pl.ds(i * 16, 16)] = zero16f
        return 0
    lax.fori_loop(0, DNCAP // 16, _zero_dn, 0)

    pltpu.async_copy(src_hbm.at[pl.ds(0, CHUNK)],
                     src_c.at[pl.ds(0, CHUNK)], sem.at[2])
    pltpu.async_copy(dst_hbm.at[pl.ds(0, CHUNK)],
                     dst_c.at[pl.ds(0, CHUNK)], sem.at[2])

    def _chunk(c, _):
        cb = (c & 1) * CHUNK
        pltpu.make_async_copy(src_hbm.at[pl.ds(0, CHUNK)],
                              src_c.at[pl.ds(cb, CHUNK)],
                              sem.at[2 + (c & 1)]).wait()
        pltpu.make_async_copy(src_hbm.at[pl.ds(0, CHUNK)],
                              dst_c.at[pl.ds(cb, CHUNK)],
                              sem.at[2 + (c & 1)]).wait()

        @pl.when(c + 1 < NCHUNK)
        def _():
            nb = ((c + 1) & 1) * CHUNK
            nbase = (c + 1) * CHUNK
            pltpu.async_copy(src_hbm.at[pl.ds(nbase, CHUNK)],
                             src_c.at[pl.ds(nb, CHUNK)],
                             sem.at[2 + ((c + 1) & 1)])
            pltpu.async_copy(dst_hbm.at[pl.ds(nbase, CHUNK)],
                             dst_c.at[pl.ds(nb, CHUNK)],
                             sem.at[2 + ((c + 1) & 1)])

        def _scan(g, nsel):
            off = cb + g * 16
            d = dst_c[pl.ds(off, 16)]
            m = (d >= lo) & (d < lo + NPW)
            cnt = plsc.all_reduce_population_count(m)[0]

            @pl.when(cnt > 0)
            def _():
                s = src_c[pl.ds(off, 16)]
                cs = plsc.cumsum(m.astype(jnp.int32))
                pos = nsel + cs - 1
                plsc.store_scatter(srcsel, [pos], s, mask=m)
                plsc.store_scatter(dstsel, [pos], d, mask=m)
            return nsel + cnt

        nsel = lax.fori_loop(0, NGRP, _scan, 0)
        ngrp = (nsel + 15) // 16

        def _issue(g):
            b16 = (g & 1) * 16
            pltpu.async_copy(wh_hbm.at[srcsel.at[pl.ds(g * 16, 16)]],
                             rows_v.at[pl.ds(b16, 16)], sem.at[g & 1])

        @pl.when(ngrp > 0)
        def _():
            _issue(0)

        def _proc(g, _):
            b16 = (g & 1) * 16
            goff = g * 16
            sv = srcsel[pl.ds(goff, 16)]
            dv = dstsel[pl.ds(goff, 16)]
            s1g = plsc.load_gather(s1_v, [sv])
            s2g = plsc.load_gather(s2_v, [dv])
            valid = lax.iota(jnp.int32, 16) < (nsel - goff)
            e16 = jnp.where(valid, jnp.exp(jnp.maximum(s1g + s2g, 0.0)), 0.0)
            dl16 = jnp.where(valid, dv - lo, 0)
            pltpu.make_async_copy(wh_hbm.at[pl.ds(0, 16)],
                                  rows_v.at[pl.ds(b16, 16)],
                                  sem.at[g & 1]).wait()

            @pl.when(g + 1 < ngrp)
            def _():
                _issue(g + 1)

            for j in range(16):
                ej = e16[j]
                dl = dl16[j]
                plsc.addupdate(denom_v.at[pl.ds(dl, 16)], ej * unit16f)
                for k in range(D // 16):
                    ko = k * 16
                    plsc.addupdate(z_v.at[dl, pl.ds(ko, 16)],
                                   ej * rows_v[b16 + j, pl.ds(ko, 16)])
            return 0

        lax.fori_loop(0, ngrp, _proc, 0)
        return 0

    lax.fori_loop(0, NCHUNK, _chunk, 0)

    def _norm(r, _):
        dnv = denom_v[pl.ds(r, 16)]
        inv = jnp.where(dnv > 0.0, 1.0 / dnv, 1.0)[0]
        for k in range(D // 16):
            ko = k * 16
            z_v[r, pl.ds(ko, 16)] = z_v[r, pl.ds(ko, 16)] * inv
        return 0
    lax.fori_loop(0, NPW, _norm, 0)

    pltpu.sync_copy(z_v.at[pl.ds(0, NPW)], out_hbm.at[pl.ds(lo, NPW)])


def _edge_phase(src, dst, s1, s2, wh):
    mesh = plsc.VectorSubcoreMesh(core_axis_name="c", subcore_axis_name="s")
    kern = pl.kernel(
        _gat_body,
        out_type=jax.ShapeDtypeStruct((NPAD, D), jnp.float32),
        mesh=mesh,
        scratch_types=[
            pltpu.VMEM((N,), jnp.float32),
            pltpu.VMEM((N,), jnp.float32),
            pltpu.VMEM((2 * CHUNK,), jnp.int32),
            pltpu.VMEM((2 * CHUNK,), jnp.int32),
            pltpu.VMEM((SELCAP,), jnp.int32),
            pltpu.VMEM((SELCAP,), jnp.int32),
            pltpu.VMEM((ZROWS, D), jnp.float32),
            pltpu.VMEM((DNCAP,), jnp.float32),
            pltpu.VMEM((32, D), jnp.float32),
            pltpu.SemaphoreType.DMA((4,)),
        ],
        compiler_params=pltpu.CompilerParams(needs_layout_passes=False),
    )
    return kern(src, dst, s1, s2, wh)


def kernel(node_features, edge_index, W1, Wa):
    src = edge_index[0]
    dst = edge_index[1]
    w1t = W1.T
    a1 = Wa[0, :D]
    a2 = Wa[0, D:]
    ap = jnp.zeros((D, 128), jnp.float32).at[:, 0].set(a1).at[:, 1].set(a2)
    wh, svec = _node_phase(node_features, w1t, ap)
    s1 = svec[:, 0]
    s2 = svec[:, 1]
    zpad = _edge_phase(src, dst, s1, s2, wh)
    return zpad[:N]


# speedup vs baseline: 4.5392x; 1.0138x over previous
import jax
import jax.numpy as jnp
from jax import lax
from jax.experimental import pallas as pl
from jax.experimental.pallas import tpu as pltpu
from jax.experimental.pallas import tpu_sc as plsc

N = 10000
E = 160000
D = 256
NW = 32
NPW = 320
NPAD = NW * NPW
ZROWS = NPW
CHUNK = 2000
NGRP = CHUNK // 16
NCHUNK = E // CHUNK
SELCAP = CHUNK + 16
DNCAP = NPW + 23


def _node_body(x_ref, w_ref, ap_ref, wh_ref, s_ref):
    wh = jnp.dot(x_ref[...], w_ref[...], preferred_element_type=jnp.float32)
    wh_ref[...] = wh
    s_ref[...] = jnp.dot(wh, ap_ref[...], preferred_element_type=jnp.float32)


def _node_phase(x, w1t, ap):
    return pl.pallas_call(
        _node_body,
        grid=(10,),
        in_specs=[
            pl.BlockSpec((1000, D), lambda i: (i, 0)),
            pl.BlockSpec((D, D), lambda i: (0, 0)),
            pl.BlockSpec((D, 128), lambda i: (0, 0)),
        ],
        out_specs=[
            pl.BlockSpec((1000, D), lambda i: (i, 0)),
            pl.BlockSpec((1000, 128), lambda i: (i, 0)),
        ],
        out_shape=[
            jax.ShapeDtypeStruct((N, D), jnp.float32),
            jax.ShapeDtypeStruct((N, 128), jnp.float32),
        ],
    )(x, w1t, ap)


def _gat_body(src_hbm, dst_hbm, s1_hbm, s2_hbm, wh_hbm, out_hbm,
              s1_v, s2_v, src_c, dst_c, srcsel, dstsel, z_v, denom_v,
              rows_v, sem):
    wid = lax.axis_index("s") * 2 + lax.axis_index("c")
    lo = wid * NPW

    pltpu.sync_copy(s1_hbm, s1_v)
    pltpu.sync_copy(s2_hbm, s2_v)

    zero16f = jnp.zeros((16,), jnp.float32)
    zero16i = jnp.zeros((16,), jnp.int32)
    unit16f = jnp.where(lax.iota(jnp.int32, 16) == 0, 1.0, 0.0)

    def _zero_sel(i, _):
        srcsel[pl.ds(i * 16, 16)] = zero16i
        dstsel[pl.ds(i * 16, 16)] = zero16i
        return 0
    lax.fori_loop(0, SELCAP // 16, _zero_sel, 0)

    def _zero_z(r, _):
        for k in range(D // 16):
            z_v[r, pl.ds(k * 16, 16)] = zero16f
        return 0
    lax.fori_loop(0, ZROWS, _zero_z, 0)

    def _zero_dn(i, _):
        denom_v[pl.ds(i * 16, 16)] = zero16f
        return 0
    lax.fori_loop(0, DNCAP // 16, _zero_dn, 0)

    pltpu.async_copy(src_hbm.at[pl.ds(0, CHUNK)],
                     src_c.at[pl.ds(0, CHUNK)], sem.at[2])
    pltpu.async_copy(dst_hbm.at[pl.ds(0, CHUNK)],
                     dst_c.at[pl.ds(0, CHUNK)], sem.at[2])

    def _chunk(c, _):
        cb = (c & 1) * CHUNK
        pltpu.make_async_copy(src_hbm.at[pl.ds(0, CHUNK)],
                              src_c.at[pl.ds(cb, CHUNK)],
                              sem.at[2 + (c & 1)]).wait()
        pltpu.make_async_copy(src_hbm.at[pl.ds(0, CHUNK)],
                              dst_c.at[pl.ds(cb, CHUNK)],
                              sem.at[2 + (c & 1)]).wait()

        @pl.when(c + 1 < NCHUNK)
        def _():
            nb = ((c + 1) & 1) * CHUNK
            nbase = (c + 1) * CHUNK
            pltpu.async_copy(src_hbm.at[pl.ds(nbase, CHUNK)],
                             src_c.at[pl.ds(nb, CHUNK)],
                             sem.at[2 + ((c + 1) & 1)])
            pltpu.async_copy(dst_hbm.at[pl.ds(nbase, CHUNK)],
                             dst_c.at[pl.ds(nb, CHUNK)],
                             sem.at[2 + ((c + 1) & 1)])

        def _scan(g, nsel):
            off = cb + g * 16
            d = dst_c[pl.ds(off, 16)]
            m = (d >= lo) & (d < lo + NPW)
            cnt = plsc.all_reduce_population_count(m)[0]

            @pl.when(cnt > 0)
            def _():
                s = src_c[pl.ds(off, 16)]
                cs = plsc.cumsum(m.astype(jnp.int32))
                pos = nsel + cs - 1
                plsc.store_scatter(srcsel, [pos], s, mask=m)
                plsc.store_scatter(dstsel, [pos], d, mask=m)
            return nsel + cnt

        nsel = lax.fori_loop(0, NGRP, _scan, 0, unroll=4)
        ngrp = (nsel + 15) // 16

        def _issue(g):
            b16 = (g & 1) * 16
            pltpu.async_copy(wh_hbm.at[srcsel.at[pl.ds(g * 16, 16)]],
                             rows_v.at[pl.ds(b16, 16)], sem.at[g & 1])

        @pl.when(ngrp > 0)
        def _():
            _issue(0)

        def _proc(g, _):
            b16 = (g & 1) * 16
            goff = g * 16
            sv = srcsel[pl.ds(goff, 16)]
            dv = dstsel[pl.ds(goff, 16)]
            s1g = plsc.load_gather(s1_v, [sv])
            s2g = plsc.load_gather(s2_v, [dv])
            valid = lax.iota(jnp.int32, 16) < (nsel - goff)
            e16 = jnp.where(valid, jnp.exp(jnp.maximum(s1g + s2g, 0.0)), 0.0)
            dl16 = jnp.where(valid, dv - lo, 0)
            pltpu.make_async_copy(wh_hbm.at[pl.ds(0, 16)],
                                  rows_v.at[pl.ds(b16, 16)],
                                  sem.at[g & 1]).wait()

            @pl.when(g + 1 < ngrp)
            def _():
                _issue(g + 1)

            for j in range(16):
                ej = e16[j]
                dl = dl16[j]
                plsc.addupdate(denom_v.at[pl.ds(dl, 16)], ej * unit16f)
                for k in range(D // 16):
                    ko = k * 16
                    plsc.addupdate(z_v.at[dl, pl.ds(ko, 16)],
                                   ej * rows_v[b16 + j, pl.ds(ko, 16)])
            return 0

        lax.fori_loop(0, ngrp, _proc, 0)
        return 0

    lax.fori_loop(0, NCHUNK, _chunk, 0)

    def _norm(r, _):
        dnv = denom_v[pl.ds(r, 16)]
        inv = jnp.where(dnv > 0.0, 1.0 / dnv, 1.0)[0]
        for k in range(D // 16):
            ko = k * 16
            z_v[r, pl.ds(ko, 16)] = z_v[r, pl.ds(ko, 16)] * inv
        return 0
    lax.fori_loop(0, NPW, _norm, 0)

    pltpu.sync_copy(z_v.at[pl.ds(0, NPW)], out_hbm.at[pl.ds(lo, NPW)])


def _edge_phase(src, dst, s1, s2, wh):
    mesh = plsc.VectorSubcoreMesh(core_axis_name="c", subcore_axis_name="s")
    kern = pl.kernel(
        _gat_body,
        out_type=jax.ShapeDtypeStruct((NPAD, D), jnp.float32),
        mesh=mesh,
        scratch_types=[
            pltpu.VMEM((N,), jnp.float32),
            pltpu.VMEM((N,), jnp.float32),
            pltpu.VMEM((2 * CHUNK,), jnp.int32),
            pltpu.VMEM((2 * CHUNK,), jnp.int32),
            pltpu.VMEM((SELCAP,), jnp.int32),
            pltpu.VMEM((SELCAP,), jnp.int32),
            pltpu.VMEM((ZROWS, D), jnp.float32),
            pltpu.VMEM((DNCAP,), jnp.float32),
            pltpu.VMEM((32, D), jnp.float32),
            pltpu.SemaphoreType.DMA((4,)),
        ],
        compiler_params=pltpu.CompilerParams(needs_layout_passes=False),
    )
    return kern(src, dst, s1, s2, wh)


def kernel(node_features, edge_index, W1, Wa):
    src = edge_index[0]
    dst = edge_index[1]
    w1t = W1.T
    a1 = Wa[0, :D]
    a2 = Wa[0, D:]
    ap = jnp.zeros((D, 128), jnp.float32).at[:, 0].set(a1).at[:, 1].set(a2)
    wh, svec = _node_phase(node_features, w1t, ap)
    s1 = svec[:, 0]
    s2 = svec[:, 1]
    zpad = _edge_phase(src, dst, s1, s2, wh)
    return zpad[:N]
